# Initial kernel scaffold; baseline (speedup 1.0000x reference)
import jax
import jax.numpy as jnp
from jax.experimental import pallas as pl


def kernel(ecg_signal, elm_input_ids, elm_attention_mask, elm_labels, signal_id_indices, proj_W, proj_b, embed_table):
    raise NotImplementedError("write your pallas kernel here")


# speedup vs baseline: 1.3713x; 1.3713x over previous
import functools

import jax
import jax.numpy as jnp
from jax import lax
from jax.experimental import pallas as pl
from jax.experimental.pallas import tpu as pltpu
from jax.experimental.pallas import tpu_sc as plsc


def _tc_proj_body(n_sig, t, ecg_ref, w_ref, b_ref, idx_ref, proj_ref, pos_ref):
    m = ecg_ref.shape[0]
    proj = jnp.dot(ecg_ref[...], w_ref[...], preferred_element_type=jnp.float32)
    proj = proj + b_ref[...]
    proj_ref[...] = proj

    idxf = idx_ref[0]
    nz = jnp.any(proj != 0, axis=1)
    valid = nz & (idxf >= 0)

    ii = lax.broadcasted_iota(jnp.int32, (m, m), 0)
    jj = lax.broadcasted_iota(jnp.int32, (m, m), 1)
    same_batch = (ii // n_sig) == (jj // n_sig)
    same_tgt = idxf[:, None] == idxf[None, :]
    later = (jj > ii) & same_batch & same_tgt & valid[None, :]
    dup_later = jnp.any(later, axis=1)
    win = valid & ~dup_later

    batch_row0 = (lax.broadcasted_iota(jnp.int32, (m,), 0) // n_sig) * t
    pos_ref[...] = jnp.where(win, batch_row0 + idxf, -1)[None, :]


def _sc_gather_body(tokens_per_w, chunk, n_sig, t,
                    ids_hbm, pos_hbm, proj_hbm, table_hbm, out_hbm,
                    ids_v, buf_v, pos_v, gsem):
    info = plsc.get_sparse_core_info()
    nc, ns = info.num_cores, info.num_subcores
    wid = lax.axis_index("c") * ns + lax.axis_index("s")
    base = wid * tokens_per_w
    b = base // t

    rows_per_w = tokens_per_w // chunk
    pltpu.sync_copy(ids_hbm.at[pl.ds(wid * rows_per_w, rows_per_w)], ids_v)
    pltpu.sync_copy(pos_hbm.at[pl.ds(b * n_sig, n_sig)], pos_v)

    def chunk_body(c, _):
        pltpu.async_copy(table_hbm.at[ids_v.at[c]], buf_v, gsem).wait()
        coff = c * chunk
        for k in range(n_sig // 16):
            rel = pos_v[pl.ds(k * 16, 16)] - (base + coff)
            for j in range(16):
                r = rel[j]

                @pl.when((r >= 0) & (r < chunk))
                def _():
                    pltpu.sync_copy(
                        proj_hbm.at[pl.ds(b * n_sig + k * 16 + j, 1)],
                        buf_v.at[pl.ds(r, 1)])
        pltpu.sync_copy(buf_v, out_hbm.at[pl.ds(base + coff, chunk)])
        return 0
    lax.fori_loop(0, rows_per_w, chunk_body, 0)


def kernel(ecg_signal, elm_input_ids, elm_attention_mask, elm_labels,
           signal_id_indices, proj_W, proj_b, embed_table):
    b_, n_sig, d_enc = ecg_signal.shape
    _, t = elm_input_ids.shape
    vocab, h = embed_table.shape
    m = b_ * n_sig

    proj, pos = pl.pallas_call(
        functools.partial(_tc_proj_body, n_sig, t),
        out_shape=(
            jax.ShapeDtypeStruct((m, h), jnp.float32),
            jax.ShapeDtypeStruct((1, m), jnp.int32),
        ),
    )(ecg_signal.reshape(m, d_enc), proj_W, proj_b.reshape(1, h),
      signal_id_indices.reshape(1, m))
    pos = pos.reshape(m)

    n_tok = b_ * t
    n_workers = 32
    tokens_per_w = n_tok // n_workers
    chunk = 32
    mesh = plsc.VectorSubcoreMesh(core_axis_name="c", subcore_axis_name="s")

    sc = pl.kernel(
        functools.partial(_sc_gather_body, tokens_per_w, chunk, n_sig, t),
        out_type=jax.ShapeDtypeStruct((n_tok, h), jnp.float32),
        mesh=mesh,
        scratch_types=[
            pltpu.VMEM((tokens_per_w // chunk, chunk), jnp.int32),
            pltpu.VMEM((chunk, h), jnp.float32),
            pltpu.VMEM((n_sig,), jnp.int32),
            pltpu.SemaphoreType.DMA,
        ],
    )

    out = sc(elm_input_ids.reshape(n_tok // chunk, chunk), pos, proj,
             embed_table)
    return out.reshape(b_, t, h)
